# baseline (speedup 1.0000x reference)
import functools

import jax
import jax.numpy as jnp
from jax import lax
from jax.experimental import pallas as pl
from jax.experimental.pallas import tpu as pltpu
from jax.experimental.pallas import tpu_sc as plsc

N_NODES = 10000
N_EDGES = 320000
D = 128

NC = 2
NS = 16
NW = NC * NS
CHUNK = 125
NCH = 80
NQ = 4
QCH = NCH // NQ
ROWS_PER_TILE = 624
TAIL_ROWS = N_NODES - NS * ROWS_PER_TILE
TAIL_OFF = NS * ROWS_PER_TILE

_DOT = (((1,), (1,)), ((), ()))


def _lin1_body(x_ref, w_ref, b_ref, o_ref):
    o_ref[...] = (
        lax.dot_general(x_ref[...], w_ref[...], _DOT,
                        preferred_element_type=jnp.float32)
        + b_ref[...]
    )


_lin1 = pl.pallas_call(
    _lin1_body,
    grid=(1,),
    in_specs=[
        pl.BlockSpec((10000, D), lambda i: (i, 0)),
        pl.BlockSpec((D, D), lambda i: (0, 0)),
        pl.BlockSpec((1, D), lambda i: (0, 0)),
    ],
    out_specs=pl.BlockSpec((10000, D), lambda i: (i, 0)),
    out_shape=jax.ShapeDtypeStruct((N_NODES, D), jnp.float32),
)


def _sc_body(h_hbm, src_hbm, dst_hbm, z_hbm, out_hbm,
             dst_v, srcq_a, srcq_b, rows_a, rows_b, acc,
             qsem_a, qsem_b, gsem_a, gsem_b, ssem_a, ssem_b):
    c = lax.axis_index("c")
    s = lax.axis_index("s")
    wid = c * NS + s

    qbufs = (srcq_a, srcq_b)
    qsems = (qsem_a, qsem_b)
    rows = (rows_a, rows_b)
    gsems = (gsem_a, gsem_b)

    pltpu.async_copy(src_hbm.at[wid, 0], srcq_a, qsem_a)
    pltpu.async_copy(dst_hbm.at[wid], dst_v, ssem_a)

    pltpu.sync_copy(z_hbm, acc.at[pl.ds(s * ROWS_PER_TILE, ROWS_PER_TILE)])
    @pl.when(s == 0)
    def _():
        pltpu.sync_copy(z_hbm.at[pl.ds(0, TAIL_ROWS)],
                        acc.at[pl.ds(TAIL_OFF, TAIL_ROWS)])

    pltpu.make_async_copy(src_hbm.at[wid, 0], srcq_a, qsem_a).wait()
    pltpu.make_async_copy(dst_hbm.at[wid], dst_v, ssem_a).wait()
    plsc.subcore_barrier()

    def fire_gather(k, qb, p):
        pltpu.async_copy(h_hbm.at[qb.at[k]], rows[p], gsems[p])

    def wait_gather(k, qb, p):
        pltpu.make_async_copy(h_hbm.at[qb.at[k]], rows[p], gsems[p]).wait()

    def step(j, k, qb, p, qb_next=None):
        if qb_next is None:
            fire_gather(k + 1, qb, 1 - p)
        elif qb_next is not False:
            fire_gather(0, qb_next, 1 - p)
        wait_gather(k, qb, p)
        pltpu.sync_copy(rows[p], acc.at[dst_v.at[j]], add=True)

    fire_gather(0, srcq_a, 0)

    for q in range(NQ):
        qb = qbufs[q % 2]
        base = QCH * q
        if q + 1 < NQ:
            nb = qbufs[(q + 1) % 2]
            nsem = qsems[(q + 1) % 2]
            pltpu.async_copy(src_hbm.at[wid, q + 1], nb, nsem)

        def pair(m, carry, qb=qb, base=base):
            k = 2 * m
            step(base + k, k, qb, 0)
            step(base + k + 1, k + 1, qb, 1)
            return carry

        lax.fori_loop(0, QCH // 2 - 1, pair, 0)
        step(base + QCH - 2, QCH - 2, qb, 0)
        if q + 1 < NQ:
            pltpu.make_async_copy(src_hbm.at[wid, q + 1], nb, nsem).wait()
            step(base + QCH - 1, QCH - 1, qb, 1, qb_next=nb)
        else:
            step(base + QCH - 1, QCH - 1, qb, 1, qb_next=False)

    plsc.subcore_barrier()

    pltpu.sync_copy(
        acc.at[pl.ds(s * ROWS_PER_TILE, ROWS_PER_TILE)],
        out_hbm.at[c].at[pl.ds(s * ROWS_PER_TILE, ROWS_PER_TILE)],
    )
    @pl.when(s == 0)
    def _():
        pltpu.sync_copy(acc.at[pl.ds(TAIL_OFF, TAIL_ROWS)],
                        out_hbm.at[c].at[pl.ds(TAIL_OFF, TAIL_ROWS)])


_sc_scatter = functools.partial(
    pl.kernel,
    out_type=jax.ShapeDtypeStruct((NC, N_NODES, D), jnp.float32),
    mesh=plsc.VectorSubcoreMesh(core_axis_name="c", subcore_axis_name="s"),
    scratch_types=[
        pltpu.VMEM((NCH, CHUNK), jnp.int32),
        pltpu.VMEM((QCH, CHUNK), jnp.int32),
        pltpu.VMEM((QCH, CHUNK), jnp.int32),
        pltpu.VMEM((CHUNK, D), jnp.float32),
        pltpu.VMEM((CHUNK, D), jnp.float32),
        pltpu.VMEM_SHARED((N_NODES, D), jnp.float32),
        pltpu.SemaphoreType.DMA,
        pltpu.SemaphoreType.DMA,
        pltpu.SemaphoreType.DMA,
        pltpu.SemaphoreType.DMA,
        pltpu.SemaphoreType.DMA,
        pltpu.SemaphoreType.DMA,
    ],
)(_sc_body)


def _lin2_body(p_ref, w_ref, b_ref, o_ref):
    a = jnp.maximum(p_ref[0] + p_ref[1], 0.0)
    o_ref[...] = (
        lax.dot_general(a, w_ref[...], _DOT,
                        preferred_element_type=jnp.float32)
        + b_ref[...]
    )


_lin2 = pl.pallas_call(
    _lin2_body,
    grid=(1,),
    in_specs=[
        pl.BlockSpec((NC, 10000, D), lambda i: (0, i, 0)),
        pl.BlockSpec((D, D), lambda i: (0, 0)),
        pl.BlockSpec((1, D), lambda i: (0, 0)),
    ],
    out_specs=pl.BlockSpec((10000, D), lambda i: (i, 0)),
    out_shape=jax.ShapeDtypeStruct((N_NODES, D), jnp.float32),
)


def kernel(x, edge_index, W1, b1, W2, b2):
    src = edge_index[0].astype(jnp.int32).reshape(NW, NQ, QCH, CHUNK)
    dst = edge_index[1].astype(jnp.int32).reshape(NW, NCH, CHUNK)
    zeros = jnp.zeros((ROWS_PER_TILE, D), jnp.float32)
    h = _lin1(x, W1, b1.reshape(1, D))
    partials = _sc_scatter(h, src, dst, zeros)
    return _lin2(partials, W2, b2.reshape(1, D))


# speedup vs baseline: 1.0038x; 1.0038x over previous
import functools

import jax
import jax.numpy as jnp
from jax import lax
from jax.experimental import pallas as pl
from jax.experimental.pallas import tpu as pltpu
from jax.experimental.pallas import tpu_sc as plsc

N_NODES = 10000
N_EDGES = 320000
D = 128

NC = 2
NS = 16
NW = NC * NS
CHUNK = 125
NCH = 80
NQ = 4
QCH = NCH // NQ
ROWS_PER_TILE = 624
TAIL_ROWS = N_NODES - NS * ROWS_PER_TILE
TAIL_OFF = NS * ROWS_PER_TILE

_DOT = (((1,), (1,)), ((), ()))


def _lin1_body(x_ref, w_ref, b_ref, o_ref):
    o_ref[...] = (
        lax.dot_general(x_ref[...], w_ref[...], _DOT,
                        preferred_element_type=jnp.float32)
        + b_ref[...]
    )


_lin1 = pl.pallas_call(
    _lin1_body,
    grid=(2,),
    in_specs=[
        pl.BlockSpec((5000, D), lambda i: (i, 0)),
        pl.BlockSpec((D, D), lambda i: (0, 0)),
        pl.BlockSpec((1, D), lambda i: (0, 0)),
    ],
    out_specs=pl.BlockSpec((5000, D), lambda i: (i, 0)),
    out_shape=jax.ShapeDtypeStruct((N_NODES, D), jnp.float32),
)


def _sc_body(h_hbm, src_hbm, dst_hbm, z_hbm, d_hbm, out_hbm,
             dst_v, srcq_a, srcq_b, rows_a, rows_b, acc,
             qsem_a, qsem_b, gsem_a, gsem_b, ssem_a, ssem_b):
    c = lax.axis_index("c")
    s = lax.axis_index("s")
    wid = c * NS + s

    qbufs = (srcq_a, srcq_b)
    qsems = (qsem_a, qsem_b)
    rows = (rows_a, rows_b)
    gsems = (gsem_a, gsem_b)

    pltpu.async_copy(src_hbm.at[wid, 0], srcq_a, qsem_a)
    pltpu.async_copy(dst_hbm.at[wid], dst_v, ssem_a)

    pltpu.sync_copy(z_hbm, acc.at[pl.ds(s * ROWS_PER_TILE, ROWS_PER_TILE)])
    @pl.when(s == 0)
    def _():
        pltpu.sync_copy(z_hbm.at[pl.ds(0, TAIL_ROWS)],
                        acc.at[pl.ds(TAIL_OFF, TAIL_ROWS)])

    pltpu.make_async_copy(src_hbm.at[wid, 0], srcq_a, qsem_a).wait()
    pltpu.make_async_copy(dst_hbm.at[wid], dst_v, ssem_a).wait()
    plsc.subcore_barrier()

    def fire_gather(k, qb, p):
        pltpu.async_copy(h_hbm.at[qb.at[k]], rows[p], gsems[p])

    def wait_gather(k, qb, p):
        pltpu.make_async_copy(d_hbm, rows[p], gsems[p]).wait()

    def step(j, k, qb, p, qb_next=None):
        if qb_next is None:
            fire_gather(k + 1, qb, 1 - p)
        elif qb_next is not False:
            fire_gather(0, qb_next, 1 - p)
        wait_gather(k, qb, p)
        pltpu.sync_copy(rows[p], acc.at[dst_v.at[j]], add=True)

    fire_gather(0, srcq_a, 0)

    for q in range(NQ):
        qb = qbufs[q % 2]
        base = QCH * q
        if q + 1 < NQ:
            nb = qbufs[(q + 1) % 2]
            nsem = qsems[(q + 1) % 2]
            pltpu.async_copy(src_hbm.at[wid, q + 1], nb, nsem)

        def pair(m, carry, qb=qb, base=base):
            k = 2 * m
            step(base + k, k, qb, 0)
            step(base + k + 1, k + 1, qb, 1)
            return carry

        lax.fori_loop(0, QCH // 2 - 1, pair, 0)
        step(base + QCH - 2, QCH - 2, qb, 0)
        if q + 1 < NQ:
            pltpu.make_async_copy(src_hbm.at[wid, q + 1], nb, nsem).wait()
            step(base + QCH - 1, QCH - 1, qb, 1, qb_next=nb)
        else:
            step(base + QCH - 1, QCH - 1, qb, 1, qb_next=False)

    plsc.subcore_barrier()

    pltpu.sync_copy(
        acc.at[pl.ds(s * ROWS_PER_TILE, ROWS_PER_TILE)],
        out_hbm.at[c].at[pl.ds(s * ROWS_PER_TILE, ROWS_PER_TILE)],
    )
    @pl.when(s == 0)
    def _():
        pltpu.sync_copy(acc.at[pl.ds(TAIL_OFF, TAIL_ROWS)],
                        out_hbm.at[c].at[pl.ds(TAIL_OFF, TAIL_ROWS)])


_sc_scatter = functools.partial(
    pl.kernel,
    out_type=jax.ShapeDtypeStruct((NC, N_NODES, D), jnp.float32),
    mesh=plsc.VectorSubcoreMesh(core_axis_name="c", subcore_axis_name="s"),
    scratch_types=[
        pltpu.VMEM((NCH, CHUNK), jnp.int32),
        pltpu.VMEM((QCH, CHUNK), jnp.int32),
        pltpu.VMEM((QCH, CHUNK), jnp.int32),
        pltpu.VMEM((CHUNK, D), jnp.float32),
        pltpu.VMEM((CHUNK, D), jnp.float32),
        pltpu.VMEM_SHARED((N_NODES, D), jnp.float32),
        pltpu.SemaphoreType.DMA,
        pltpu.SemaphoreType.DMA,
        pltpu.SemaphoreType.DMA,
        pltpu.SemaphoreType.DMA,
        pltpu.SemaphoreType.DMA,
        pltpu.SemaphoreType.DMA,
    ],
)(_sc_body)


def _lin2_body(p_ref, w_ref, b_ref, o_ref):
    a = jnp.maximum(p_ref[0] + p_ref[1], 0.0)
    o_ref[...] = (
        lax.dot_general(a, w_ref[...], _DOT,
                        preferred_element_type=jnp.float32)
        + b_ref[...]
    )


_lin2 = pl.pallas_call(
    _lin2_body,
    grid=(2,),
    in_specs=[
        pl.BlockSpec((NC, 5000, D), lambda i: (0, i, 0)),
        pl.BlockSpec((D, D), lambda i: (0, 0)),
        pl.BlockSpec((1, D), lambda i: (0, 0)),
    ],
    out_specs=pl.BlockSpec((5000, D), lambda i: (i, 0)),
    out_shape=jax.ShapeDtypeStruct((N_NODES, D), jnp.float32),
)


def kernel(x, edge_index, W1, b1, W2, b2):
    src = edge_index[0].astype(jnp.int32).reshape(NW, NQ, QCH, CHUNK)
    dst = edge_index[1].astype(jnp.int32).reshape(NW, NCH, CHUNK)
    zeros = jnp.zeros((ROWS_PER_TILE, D), jnp.float32)
    h = _lin1(x, W1, b1.reshape(1, D))
    dummy = jnp.zeros((CHUNK, D), jnp.float32)
    partials = _sc_scatter(h, src, dst, zeros, dummy)
    return _lin2(partials, W2, b2.reshape(1, D))


# speedup vs baseline: 1.0121x; 1.0082x over previous
import functools

import jax
import jax.numpy as jnp
from jax import lax
from jax.experimental import pallas as pl
from jax.experimental.pallas import tpu as pltpu
from jax.experimental.pallas import tpu_sc as plsc

N_NODES = 10000
N_EDGES = 320000
D = 128

NC = 2
NS = 16
NW = NC * NS
CHUNK = 125
NCH = 80
NQ = 4
QCH = NCH // NQ
ROWS_PER_TILE = 624
TAIL_ROWS = N_NODES - NS * ROWS_PER_TILE
TAIL_OFF = NS * ROWS_PER_TILE

_DOT = (((1,), (1,)), ((), ()))


def _lin1_body(x_ref, w_ref, b_ref, o_ref):
    o_ref[...] = (
        lax.dot_general(x_ref[...], w_ref[...], _DOT,
                        preferred_element_type=jnp.float32)
        + b_ref[...]
    )


_lin1 = pl.pallas_call(
    _lin1_body,
    grid=(2,),
    in_specs=[
        pl.BlockSpec((5000, D), lambda i: (i, 0)),
        pl.BlockSpec((D, D), lambda i: (0, 0)),
        pl.BlockSpec((1, D), lambda i: (0, 0)),
    ],
    out_specs=pl.BlockSpec((5000, D), lambda i: (i, 0)),
    out_shape=jax.ShapeDtypeStruct((N_NODES, D), jnp.float32),
)


def _sc_body(h_hbm, src_hbm, dst_hbm, z_hbm, out_hbm,
             dst_v, srcq_a, srcq_b, rows_a, rows_b, acc,
             qsem_a, qsem_b, gsem_a, gsem_b, ssem_a, ssem_b):
    c = lax.axis_index("c")
    s = lax.axis_index("s")
    wid = c * NS + s

    qbufs = (srcq_a, srcq_b)
    qsems = (qsem_a, qsem_b)
    rows = (rows_a, rows_b)
    gsems = (gsem_a, gsem_b)

    pltpu.async_copy(src_hbm.at[wid, 0], srcq_a, qsem_a)
    pltpu.async_copy(dst_hbm.at[wid], dst_v, ssem_a)

    pltpu.sync_copy(z_hbm, acc.at[pl.ds(s * ROWS_PER_TILE, ROWS_PER_TILE)])
    @pl.when(s == 0)
    def _():
        pltpu.sync_copy(z_hbm.at[pl.ds(0, TAIL_ROWS)],
                        acc.at[pl.ds(TAIL_OFF, TAIL_ROWS)])

    pltpu.make_async_copy(src_hbm.at[wid, 0], srcq_a, qsem_a).wait()
    pltpu.make_async_copy(dst_hbm.at[wid], dst_v, ssem_a).wait()
    plsc.subcore_barrier()

    def fire_gather(k, qb, p):
        pltpu.async_copy(h_hbm.at[qb.at[k]], rows[p], gsems[p])

    def wait_gather(k, qb, p):
        pltpu.make_async_copy(h_hbm.at[qb.at[k]], rows[p], gsems[p]).wait()

    def step(j, k, qb, p, qb_next=None):
        if qb_next is None:
            fire_gather(k + 1, qb, 1 - p)
        elif qb_next is not False:
            fire_gather(0, qb_next, 1 - p)
        wait_gather(k, qb, p)
        pltpu.sync_copy(rows[p], acc.at[dst_v.at[j]], add=True)

    fire_gather(0, srcq_a, 0)

    for q in range(NQ):
        qb = qbufs[q % 2]
        base = QCH * q
        if q + 1 < NQ:
            nb = qbufs[(q + 1) % 2]
            nsem = qsems[(q + 1) % 2]
            pltpu.async_copy(src_hbm.at[wid, q + 1], nb, nsem)

        def pair(m, carry, qb=qb, base=base):
            k = 2 * m
            step(base + k, k, qb, 0)
            step(base + k + 1, k + 1, qb, 1)
            return carry

        lax.fori_loop(0, QCH // 2 - 1, pair, 0)
        step(base + QCH - 2, QCH - 2, qb, 0)
        if q + 1 < NQ:
            pltpu.make_async_copy(src_hbm.at[wid, q + 1], nb, nsem).wait()
            step(base + QCH - 1, QCH - 1, qb, 1, qb_next=nb)
        else:
            step(base + QCH - 1, QCH - 1, qb, 1, qb_next=False)

    plsc.subcore_barrier()

    pltpu.sync_copy(
        acc.at[pl.ds(s * ROWS_PER_TILE, ROWS_PER_TILE)],
        out_hbm.at[c].at[pl.ds(s * ROWS_PER_TILE, ROWS_PER_TILE)],
    )
    @pl.when(s == 0)
    def _():
        pltpu.sync_copy(acc.at[pl.ds(TAIL_OFF, TAIL_ROWS)],
                        out_hbm.at[c].at[pl.ds(TAIL_OFF, TAIL_ROWS)])


_sc_scatter = functools.partial(
    pl.kernel,
    out_type=jax.ShapeDtypeStruct((NC, N_NODES, D), jnp.float32),
    mesh=plsc.VectorSubcoreMesh(core_axis_name="c", subcore_axis_name="s"),
    scratch_types=[
        pltpu.VMEM((NCH, CHUNK), jnp.int32),
        pltpu.VMEM((QCH, CHUNK), jnp.int32),
        pltpu.VMEM((QCH, CHUNK), jnp.int32),
        pltpu.VMEM((CHUNK, D), jnp.float32),
        pltpu.VMEM((CHUNK, D), jnp.float32),
        pltpu.VMEM_SHARED((N_NODES, D), jnp.float32),
        pltpu.SemaphoreType.DMA,
        pltpu.SemaphoreType.DMA,
        pltpu.SemaphoreType.DMA,
        pltpu.SemaphoreType.DMA,
        pltpu.SemaphoreType.DMA,
        pltpu.SemaphoreType.DMA,
    ],
)(_sc_body)


def _lin2_body(p_ref, w_ref, b_ref, o_ref):
    a = jnp.maximum(p_ref[0] + p_ref[1], 0.0)
    o_ref[...] = (
        lax.dot_general(a, w_ref[...], _DOT,
                        preferred_element_type=jnp.float32)
        + b_ref[...]
    )


_lin2 = pl.pallas_call(
    _lin2_body,
    grid=(2,),
    in_specs=[
        pl.BlockSpec((NC, 5000, D), lambda i: (0, i, 0)),
        pl.BlockSpec((D, D), lambda i: (0, 0)),
        pl.BlockSpec((1, D), lambda i: (0, 0)),
    ],
    out_specs=pl.BlockSpec((5000, D), lambda i: (i, 0)),
    out_shape=jax.ShapeDtypeStruct((N_NODES, D), jnp.float32),
)


def kernel(x, edge_index, W1, b1, W2, b2):
    src = edge_index[0].astype(jnp.int32).reshape(NW, NQ, QCH, CHUNK)
    dst = edge_index[1].astype(jnp.int32).reshape(NW, NCH, CHUNK)
    zeros = jnp.zeros((ROWS_PER_TILE, D), jnp.float32)
    h = _lin1(x, W1, b1.reshape(1, D))
    partials = _sc_scatter(h, src, dst, zeros)
    return _lin2(partials, W2, b2.reshape(1, D))
